# baseline (speedup 1.0000x reference)
import functools

import jax
import jax.numpy as jnp
from jax import lax
from jax.experimental import pallas as pl
from jax.experimental.pallas import tpu as pltpu
from jax.experimental.pallas import tpu_sc as plsc

NUM_USERS = 1000000
NUM_ITEMS = 1000000
EMBED_K = 16
BATCH = 16384

_NC = 2
_NS = 16
_NW = _NC * _NS
_BPW = BATCH // _NW
_L = 16
_RPG = 128 // EMBED_K
_CHUNK = 256


def _body(uidx_hbm, vidx_hbm, w_hbm, h_hbm, out_hbm,
          uidx_v, vidx_v, ugidx_v, vgidx_v, ugrp_v, vgrp_v, out_v,
          sem_u, sem_v):
    wid = lax.axis_index("s") * _NC + lax.axis_index("c")
    base = wid * _BPW

    pltpu.sync_copy(uidx_hbm.at[pl.ds(base, _BPW)], uidx_v)
    pltpu.sync_copy(vidx_hbm.at[pl.ds(base, _BPW)], vidx_v)

    def gidx(i, _):
        u = uidx_v[pl.ds(i * _L, _L)]
        v = vidx_v[pl.ds(i * _L, _L)]
        ugidx_v[pl.ds(i * _L, _L)] = u // _RPG
        vgidx_v[pl.ds(i * _L, _L)] = v // _RPG
        return 0

    lax.fori_loop(0, _BPW // _L, gidx, 0)

    lanes = lax.iota(jnp.int32, _L)

    def chunk(c, _):
        cp_u = pltpu.make_async_copy(
            w_hbm.at[ugidx_v.at[pl.ds(c * _CHUNK, _CHUNK)]], ugrp_v, sem_u)
        cp_v = pltpu.make_async_copy(
            h_hbm.at[vgidx_v.at[pl.ds(c * _CHUNK, _CHUNK)]], vgrp_v, sem_v)
        cp_u.start()
        cp_v.start()
        cp_u.wait()
        cp_v.wait()

        def group(g, _):
            i = c * _CHUNK + g * _L
            rows = g * _L + lanes
            uoff = (uidx_v[pl.ds(i, _L)] % _RPG) * EMBED_K
            voff = (vidx_v[pl.ds(i, _L)] % _RPG) * EMBED_K
            acc = jnp.zeros((_L,), jnp.float32)
            for k in range(EMBED_K):
                u = plsc.load_gather(ugrp_v, [rows, uoff + k])
                v = plsc.load_gather(vgrp_v, [rows, voff + k])
                acc = acc + u * v
            inner = 1.0 / (1.0 + jnp.exp(-acc))
            pred = 1.0 / (1.0 + jnp.exp(-inner))
            out_v[pl.ds(i, _L)] = pred
            return 0

        lax.fori_loop(0, _CHUNK // _L, group, 0)
        return 0

    lax.fori_loop(0, _BPW // _CHUNK, chunk, 0)

    pltpu.sync_copy(out_v, out_hbm.at[pl.ds(base, _BPW)])


@jax.jit
def _run(uidx, vidx, w, h):
    mesh = plsc.VectorSubcoreMesh(core_axis_name="c", subcore_axis_name="s")
    f = pl.kernel(
        _body,
        mesh=mesh,
        out_type=jax.ShapeDtypeStruct((BATCH,), jnp.float32),
        compiler_params=pltpu.CompilerParams(needs_layout_passes=False),
        scratch_types=[
            pltpu.VMEM((_BPW,), jnp.int32),
            pltpu.VMEM((_BPW,), jnp.int32),
            pltpu.VMEM((_BPW,), jnp.int32),
            pltpu.VMEM((_BPW,), jnp.int32),
            pltpu.VMEM((_CHUNK, 128), jnp.float32),
            pltpu.VMEM((_CHUNK, 128), jnp.float32),
            pltpu.VMEM((_BPW,), jnp.float32),
            pltpu.SemaphoreType.DMA,
            pltpu.SemaphoreType.DMA,
        ],
    )
    return f(uidx, vidx, w, h)


def kernel(x, W, H):
    uidx = x[:, 0]
    vidx = x[:, 1]
    w = W.reshape(NUM_USERS // _RPG, 128)
    h = H.reshape(NUM_ITEMS // _RPG, 128)
    return _run(uidx, vidx, w, h)
